# baseline (speedup 1.0000x reference)
import functools

import jax
import jax.numpy as jnp
from jax import lax
from jax.experimental import pallas as pl
from jax.experimental.pallas import tpu as pltpu
from jax.experimental.pallas import tpu_sc as plsc

N_NODES = 50000
E_EDGES = 800000
H1 = 32
H2 = 32
PACK = 4
NL2D = 196
NPAD = NL2D * 128
NLIST = 25000

NSC = 2
NTILE = 16
E_PAD = 802816
E_IDX = 800768
FC = 781
TAIL0 = FC * 8 * 128
ACC_R = 50432
ACC_RPT = ACC_R // NTILE
DUMMY = 50300

_SC_MESH = plsc.VectorSubcoreMesh(core_axis_name="c", subcore_axis_name="s")


def _leaky(x):
    return jnp.where(x > 0, x, 0.2 * x)


def _bd(w):
    return jnp.kron(jnp.eye(PACK, dtype=w.dtype), w)


def _bt(b):
    return jnp.tile(b, PACK)[None, :]


def _mlp_out_body(gs, gd, ef, w1s, w1d, w1e, b1, w2, b2, w3, b3, w4, b4,
                  w5a, b5a, w5b, b5b, w5c, b5c, w5d, b5d, oa, ob, oc, od):
    x = gs[...] @ w1s[...] + gd[...] @ w1d[...] + ef[...] @ w1e[...] + b1[...]
    x = _leaky(x)
    x = _leaky(x @ w2[...] + b2[...])
    x = _leaky(x @ w3[...] + b3[...])
    x = _leaky(x @ w4[...] + b4[...])
    oa[...] = x @ w5a[...] + b5a[...]
    ob[...] = x @ w5b[...] + b5b[...]
    oc[...] = x @ w5c[...] + b5c[...]
    od[...] = x @ w5d[...] + b5d[...]


def _edge_mlp_out(gs, gd, ef, ps):
    e4 = E_EDGES // PACK
    (w1, b1), (w2, b2), (w3, b3), (w4, b4), (w5, b5) = ps
    args = (gs.reshape(-1, 128), gd.reshape(-1, 128), ef.reshape(e4, 64),
            _bd(w1[:32]), _bd(w1[32:64]), _bd(w1[64:80]), _bt(b1),
            _bd(w2), _bt(b2), _bd(w3), _bt(b3), _bd(w4), _bt(b4),
            _bd(w5[:, 0:16]), _bt(b5[0:16]), _bd(w5[:, 16:32]), _bt(b5[16:32]),
            _bd(w5[:, 32:48]), _bt(b5[32:48]), _bd(w5[:, 48:64]), _bt(b5[48:64]))
    R = 2000
    in_specs = [
        pl.BlockSpec((R, 128), lambda i: (i, 0)),
        pl.BlockSpec((R, 128), lambda i: (i, 0)),
        pl.BlockSpec((R, 64), lambda i: (i, 0)),
    ] + [pl.BlockSpec(a.shape, lambda i: (0, 0)) for a in args[3:]]
    ospec = [pl.BlockSpec((R, 64), lambda i: (i, 0))] * 4
    oshape = [jax.ShapeDtypeStruct((e4, 64), jnp.float32)] * 4
    qs = pl.pallas_call(
        _mlp_out_body,
        grid=(e4 // R,),
        in_specs=in_specs,
        out_specs=ospec,
        out_shape=oshape,
    )(*args)
    return [q.reshape(-1, 16) for q in qs]


def _mlp_in_body(gs, gd, ef, w1s, w1d, w1e, b1, w2, b2, w3, b3,
                 w5k, b5k, w5k16, b5k16, w5a, b5a, w5b, b5b, w5c, b5c,
                 f1ao, f1bo, f2o):
    x = gs[...] @ w1s[...] + gd[...] @ w1d[...] + ef[...] @ w1e[...] + b1[...]
    x = _leaky(x)
    x = _leaky(x @ w2[...] + b2[...])
    x = _leaky(x @ w3[...] + b3[...])
    k32 = 1.0 / (1.0 + jnp.exp(-(x @ w5k[...] + b5k[...])))
    k16 = 1.0 / (1.0 + jnp.exp(-(x @ w5k16[...] + b5k16[...])))
    f1ao[...] = (x @ w5a[...] + b5a[...]) * k16
    f1bo[...] = (x @ w5b[...] + b5b[...]) * k16
    f2o[...] = (x @ w5c[...] + b5c[...]) * k32


def _edge_mlp_in(gs, gd, ef, ps):
    e4 = E_EDGES // PACK
    (w1, b1), (w2, b2), (w3, b3), (w5, b5) = ps
    w5k = jnp.tile(w5[:, 0:1], (1, 32))
    b5k = jnp.tile(b5[0:1], 32)
    w5k16 = jnp.tile(w5[:, 0:1], (1, 16))
    b5k16 = jnp.tile(b5[0:1], 16)
    args = (gs.reshape(-1, 128), gd.reshape(-1, 128), ef.reshape(e4, 64),
            _bd(w1[:32]), _bd(w1[32:64]), _bd(w1[64:80]), _bt(b1),
            _bd(w2), _bt(b2), _bd(w3), _bt(b3),
            _bd(w5k), _bt(b5k), _bd(w5k16), _bt(b5k16),
            _bd(w5[:, 1:17]), _bt(b5[1:17]),
            _bd(w5[:, 17:33]), _bt(b5[17:33]),
            _bd(w5[:, 33:65]), _bt(b5[33:65]))
    R = 2000
    in_specs = [
        pl.BlockSpec((R, 128), lambda i: (i, 0)),
        pl.BlockSpec((R, 128), lambda i: (i, 0)),
        pl.BlockSpec((R, 64), lambda i: (i, 0)),
    ] + [pl.BlockSpec(a.shape, lambda i: (0, 0)) for a in args[3:]]
    f1a, f1b, f2 = pl.pallas_call(
        _mlp_in_body,
        grid=(e4 // R,),
        in_specs=in_specs,
        out_specs=[pl.BlockSpec((R, 64), lambda i: (i, 0)),
                   pl.BlockSpec((R, 64), lambda i: (i, 0)),
                   pl.BlockSpec((R, 128), lambda i: (i, 0))],
        out_shape=[jax.ShapeDtypeStruct((e4, 64), jnp.float32),
                   jax.ShapeDtypeStruct((e4, 64), jnp.float32),
                   jax.ShapeDtypeStruct((e4, 128), jnp.float32)],
    )(*args)
    return (f1a.reshape(-1, 16), f1b.reshape(-1, 16), f2.reshape(-1, 32))


def _sc_segsum16(parts, dst2d, zrows, dep):
    P = len(parts) // 2
    GR = 8
    NCH = (FC + NTILE - 1) // NTILE

    @functools.partial(
        pl.kernel, mesh=_SC_MESH,
        out_type=[jax.ShapeDtypeStruct((ACC_R, 16), jnp.float32)] * (2 * P),
        scratch_types=[
            pltpu.VMEM((GR * 128, 16), jnp.float32),
            pltpu.VMEM((9, 128), jnp.int32),
            pltpu.VMEM_SHARED((ACC_R, 16), jnp.float32),
        ],
        compiler_params=pltpu.CompilerParams(use_tc_tiling_on_sc=False),
    )
    def k(*refs):
        srcs = refs[:2 * P]
        dst_hbm, z_hbm = refs[2 * P:2 * P + 2]
        dep_h = refs[2 * P + 2]
        outs = refs[2 * P + 3:2 * P + 3 + 2 * P]
        buf, idx, acc = refs[2 * P + 3 + 2 * P:]
        cid = lax.axis_index("c")
        sid = lax.axis_index("s")
        pltpu.sync_copy(dep_h.at[pl.ds(0, 8)], buf.at[pl.ds(0, 8)])

        def run(src_hbm, out_hbm):
            pltpu.sync_copy(z_hbm, acc.at[pl.ds(sid * ACC_RPT, ACC_RPT)])
            plsc.subcore_barrier()

            def body(j, carry):
                c = j * NTILE + sid

                @pl.when(c < FC)
                def _():
                    row0 = c * GR
                    pltpu.sync_copy(dst_hbm.at[pl.ds(row0, GR)],
                                    idx.at[pl.ds(0, GR)])
                    pltpu.sync_copy(src_hbm.at[pl.ds(row0 * 128, GR * 128)],
                                    buf)
                    for j2 in range(GR):
                        pltpu.sync_copy(buf.at[pl.ds(j2 * 128, 128)],
                                        acc.at[idx.at[j2]], add=True)

                return carry

            lax.fori_loop(0, NCH, body, 0)

            @pl.when(sid == 0)
            def _():
                pltpu.sync_copy(dst_hbm.at[pl.ds(FC * 8, 2)],
                                idx.at[pl.ds(0, 2)])
                pltpu.sync_copy(src_hbm.at[pl.ds(TAIL0, 256)],
                                buf.at[pl.ds(0, 256)])
                for j2 in range(2):
                    pltpu.sync_copy(buf.at[pl.ds(j2 * 128, 128)],
                                    acc.at[idx.at[j2]], add=True)
            plsc.subcore_barrier()
            pltpu.sync_copy(acc.at[pl.ds(sid * ACC_RPT, ACC_RPT)],
                            out_hbm.at[pl.ds(sid * ACC_RPT, ACC_RPT)])

        for p in range(P):
            @pl.when(cid == 0)
            def _(p=p):
                run(srcs[p], outs[p])

            @pl.when(cid == 1)
            def _(p=p):
                run(srcs[P + p], outs[P + p])

    return k(*parts, dst2d, zrows, dep)


def _sc_gather_edges(nf, idxs):
    GR = 8
    NCH = (FC + 31) // 32

    @functools.partial(
        pl.kernel, mesh=_SC_MESH,
        out_type=[jax.ShapeDtypeStruct((E_PAD, 32), jnp.float32)] * 4,
        scratch_types=[
            pltpu.VMEM((GR * 128, 32), jnp.float32),
            pltpu.VMEM((11, 128), jnp.int32),
            pltpu.SemaphoreType.DMA,
        ],
        compiler_params=pltpu.CompilerParams(use_tc_tiling_on_sc=False),
    )
    def k(nf_hbm, i0, i1, i2, i3, o0, o1, o2, o3, buf, idx, sem):
        cid = lax.axis_index("c")
        sid = lax.axis_index("s")
        wid = sid * NSC + cid

        def stream(idx_hbm, out_hbm, tailw):
            def body(j, carry):
                c = j * 32 + wid

                @pl.when(c < FC)
                def _():
                    row0 = c * GR
                    pltpu.sync_copy(idx_hbm.at[pl.ds(row0, GR)],
                                    idx.at[pl.ds(0, GR)])
                    for j2 in range(GR):
                        pltpu.async_copy(
                            nf_hbm.at[idx.at[j2]],
                            buf.at[pl.ds(j2 * 128, 128)], sem).wait()
                    pltpu.sync_copy(buf,
                                    out_hbm.at[pl.ds(row0 * 128, GR * 128)])

                return carry

            lax.fori_loop(0, NCH, body, 0)

            @pl.when(wid == tailw)
            def _():
                pltpu.sync_copy(idx_hbm.at[pl.ds(FC * 8, 2)],
                                idx.at[pl.ds(0, 2)])
                for j2 in range(2):
                    pltpu.async_copy(
                        nf_hbm.at[idx.at[j2]],
                        buf.at[pl.ds(j2 * 128, 128)], sem).wait()
                pltpu.sync_copy(buf.at[pl.ds(0, 256)],
                                out_hbm.at[pl.ds(TAIL0, 256)])

        stream(i0, o0, 0)
        stream(i1, o1, 1)
        stream(i2, o2, 2)
        stream(i3, o3, 3)

    return k(nf, *idxs)


def _sc_scatter_final(eia, eib, eoa, eob, inp_l, out_l, zrows16):
    @functools.partial(
        pl.kernel, mesh=_SC_MESH,
        out_type=jax.ShapeDtypeStruct((N_NODES, 32), jnp.float32),
        scratch_types=[
            pltpu.VMEM((128, 16), jnp.float32),
            pltpu.VMEM((12, 128), jnp.int32),
            pltpu.VMEM((128,), jnp.int32),
            pltpu.VMEM_SHARED((ACC_R, 16), jnp.float32),
        ],
        compiler_params=pltpu.CompilerParams(use_tc_tiling_on_sc=False),
    )
    def k(eia_h, eib_h, eoa_h, eob_h, inp_h, out_h, z_hbm, onf,
          buf, idx, lbuf, acc):
        cid = lax.axis_index("c")
        sid = lax.axis_index("s")
        pltpu.sync_copy(z_hbm, acc.at[pl.ds(sid * ACC_RPT, ACC_RPT)])
        plsc.subcore_barrier()

        def scat(rows_hbm, idx_hbm):
            def body(j, carry):
                c = j * NTILE + sid

                @pl.when(c < NL2D)
                def _():
                    _load_list_row(idx_hbm, lbuf, idx, 0, c, DUMMY)
                    pltpu.sync_copy(rows_hbm.at[pl.ds(c * 128, 128)], buf)
                    pltpu.sync_copy(buf, acc.at[idx.at[0]], add=True)

                return carry

            lax.fori_loop(0, (NL2D + NTILE - 1) // NTILE, body, 0)

        @pl.when(cid == 0)
        def _():
            scat(eia_h, inp_h)
            scat(eoa_h, out_h)

        @pl.when(cid == 1)
        def _():
            scat(eib_h, inp_h)
            scat(eob_h, out_h)

        plsc.subcore_barrier()
        row0 = sid * ACC_RPT
        tail = N_NODES - 15 * ACC_RPT

        @pl.when(sid < 15)
        def _():
            pltpu.sync_copy(acc.at[pl.ds(row0, ACC_RPT)],
                            onf.at[pl.ds(row0, ACC_RPT), pl.ds(cid * 16, 16)])

        @pl.when(sid == 15)
        def _():
            pltpu.sync_copy(acc.at[pl.ds(row0, tail)],
                            onf.at[pl.ds(row0, tail), pl.ds(cid * 16, 16)])

    return k(eia, eib, eoa, eob, inp_l, out_l, zrows16)


MR = ACC_R // 32


def _sc_segmax(f2, dst1d, neg, dep):
    CH = 4096
    NCHM = E_EDGES // CH

    @functools.partial(
        pl.kernel, mesh=_SC_MESH,
        out_type=jax.ShapeDtypeStruct((ACC_R, 32), jnp.float32),
        scratch_types=[
            pltpu.VMEM((CH,), jnp.int32),
            pltpu.VMEM((CH,), jnp.int32),
            pltpu.VMEM((CH,), jnp.int32),
            pltpu.VMEM((128, 32), jnp.float32),
            pltpu.VMEM((MR, 32), jnp.float32),
            pltpu.SemaphoreType.DMA,
        ],
        compiler_params=pltpu.CompilerParams(use_tc_tiling_on_sc=False),
    )
    def k(f2_hbm, dst_hbm, neg_hbm, dep_h, out_hbm, dstb, eids, lrows,
          f2b, acc, sem):
        cid = lax.axis_index("c")
        sid = lax.axis_index("s")
        wid = sid * NSC + cid
        lo = wid * MR
        hi = lo + MR
        pltpu.sync_copy(dep_h.at[pl.ds(0, 8)], f2b.at[pl.ds(0, 8), pl.ds(0, 8)])
        pltpu.sync_copy(neg_hbm, acc)
        ii = lax.iota(jnp.int32, 16)

        def chunk(c, carry, chn=CH):
            base = c * CH
            pltpu.sync_copy(dst_hbm.at[pl.ds(base, chn)],
                            dstb.at[pl.ds(0, chn)])

            def scan(j, cnt):
                for l in range(8):
                    v = dstb[pl.ds(j * 128 + l * 16, 16)]
                    m = (v >= lo) & (v < hi)
                    mi = m.astype(jnp.int32)
                    pref = plsc.cumsum(mi)
                    pos = cnt + pref - mi
                    eidv = base + j * 128 + l * 16 + ii
                    plsc.store_scatter(eids, [pos], eidv, mask=m)
                    plsc.store_scatter(lrows, [pos], v - lo, mask=m)
                    cnt = cnt + jnp.sum(mi)
                return cnt

            cnt = lax.fori_loop(0, chn // 128, scan, 0)

            def drain(g, carry2):
                @pl.when(g * 128 < cnt)
                def _():
                    pltpu.async_copy(
                        f2_hbm.at[eids.at[pl.ds(g * 128, 128)]], f2b,
                        sem).wait()

                    def upd(q, carry3):
                        lrv = lrows[pl.ds(g * 128 + q * 16, 16)]
                        for e in range(16):
                            gidx = g * 128 + q * 16 + e

                            @pl.when(gidx < cnt)
                            def _(e=e, lrv=lrv, gidx=gidx, q=q):
                                row = jnp.sum(
                                    jnp.where(ii == e, lrv, 0))
                                fr = q * 16 + e
                                a0 = acc[row, pl.ds(0, 16)]
                                v0 = f2b[fr, pl.ds(0, 16)]
                                acc[row, pl.ds(0, 16)] = jnp.maximum(a0, v0)
                                a1 = acc[row, pl.ds(16, 16)]
                                v1 = f2b[fr, pl.ds(16, 16)]
                                acc[row, pl.ds(16, 16)] = jnp.maximum(a1, v1)

                        return carry3

                    lax.fori_loop(0, 8, upd, 0)

                return carry2

            lax.fori_loop(0, chn // 128, drain, 0)
            return carry

        lax.fori_loop(0, NCHM, chunk, 0)
        chunk(NCHM, 0, chn=E_EDGES - NCHM * CH)
        pltpu.sync_copy(acc, out_hbm.at[pl.ds(lo, MR)])

    return k(f2, dst1d, neg, dep)


def _sc_gather_nodes(tables, inp_l, out_l):
    outs_t = [jax.ShapeDtypeStruct((NL2D * 128, w), jnp.float32)
              for (_, _, w) in tables]

    @functools.partial(
        pl.kernel, mesh=_SC_MESH,
        out_type=outs_t,
        scratch_types=[
            pltpu.VMEM((1024, 32), jnp.float32),
            pltpu.VMEM((1024, 16), jnp.float32),
            pltpu.VMEM((1024, 8), jnp.float32),
            pltpu.VMEM((13, 128), jnp.int32),
            pltpu.VMEM((14, 128), jnp.int32),
            pltpu.VMEM((128,), jnp.int32),
            pltpu.SemaphoreType.DMA,
        ],
        compiler_params=pltpu.CompilerParams(use_tc_tiling_on_sc=False),
    )
    def k(*refs):
        nt = len(tables)
        tabs = refs[:nt]
        inp_h, out_h = refs[nt], refs[nt + 1]
        outs = refs[nt + 2:nt + 2 + nt]
        b32, b16, b8, idxa, idxb, lbuf, sem = refs[nt + 2 + nt:]
        cid = lax.axis_index("c")
        sid = lax.axis_index("s")
        wid = sid * NSC + cid

        @pl.when(wid < 25)
        def _():
            row0 = wid * 8
            for j in range(8):
                @pl.when(row0 + j < NL2D)
                def _(j=j):
                    _load_list_row(inp_h, lbuf, idxa, j, row0 + j, 0)
                    _load_list_row(out_h, lbuf, idxb, j, row0 + j, 0)

            for t, (_, which, w) in enumerate(tables):
                buf = {32: b32, 16: b16, 8: b8}[w]
                idx = idxa if which == 0 else idxb
                for j in range(8):
                    @pl.when(row0 + j < NL2D)
                    def _(j=j, t=t, buf=buf, idx=idx):
                        pltpu.async_copy(
                            tabs[t].at[idx.at[j]],
                            buf.at[pl.ds(j * 128, 128)], sem).wait()

                @pl.when(row0 + 8 <= NL2D)
                def _(t=t, buf=buf):
                    pltpu.sync_copy(buf,
                                    outs[t].at[pl.ds(row0 * 128, 1024)])

                @pl.when(row0 + 8 > NL2D)
                def _(t=t, buf=buf):
                    pltpu.sync_copy(
                        buf.at[pl.ds(0, (NL2D % 8) * 128)],
                        outs[t].at[pl.ds(row0 * 128, (NL2D % 8) * 128)])

    return k(*[t for (t, _, _) in tables], inp_l, out_l)


def _load_list_row(nl_hbm, lbuf, idx, row, c, fill):
    ii = lax.iota(jnp.int32, 16)

    @pl.when(c < NL2D - 1)
    def _():
        pltpu.sync_copy(nl_hbm.at[pl.ds(c * 128, 128)], lbuf)

    @pl.when(c == NL2D - 1)
    def _():
        pltpu.sync_copy(nl_hbm.at[pl.ds((NL2D - 1) * 128, NLIST % 128)],
                        lbuf.at[pl.ds(0, NLIST % 128)])

    for l in range(8):
        v = lbuf[pl.ds(l * 16, 16)]
        vv = jnp.where(c * 128 + l * 16 + ii < NLIST, v, fill)
        idx[row, pl.ds(l * 16, 16)] = vv


def _sc_counts(dst2d, inp_l, out_l, pats, zrows8, dep):
    GR = 8
    NCH = (FC + NTILE - 1) // NTILE
    NCHN = (NL2D + NTILE - 1) // NTILE

    @functools.partial(
        pl.kernel, mesh=_SC_MESH,
        out_type=[jax.ShapeDtypeStruct((ACC_R, 8), jnp.float32)] * 2,
        scratch_types=[
            pltpu.VMEM((384, 8), jnp.float32),
            pltpu.VMEM((10, 128), jnp.int32),
            pltpu.VMEM((128,), jnp.int32),
            pltpu.VMEM_SHARED((ACC_R, 8), jnp.float32),
        ],
        compiler_params=pltpu.CompilerParams(use_tc_tiling_on_sc=False),
    )
    def k(dst_hbm, inp_hbm, outl_hbm, pat_hbm, z_hbm, dep_h,
          o0_hbm, o1_hbm, patb, idx, lbuf, acc):
        cid = lax.axis_index("c")
        sid = lax.axis_index("s")
        pltpu.sync_copy(dep_h.at[pl.ds(0, 8), pl.ds(0, 8)],
                        patb.at[pl.ds(0, 8)])
        pltpu.sync_copy(pat_hbm, patb)
        pltpu.sync_copy(z_hbm, acc.at[pl.ds(sid * ACC_RPT, ACC_RPT)])
        plsc.subcore_barrier()

        @pl.when(cid == 0)
        def _():
            def body(j, carry):
                c = j * NTILE + sid

                @pl.when(c < FC)
                def _():
                    pltpu.sync_copy(dst_hbm.at[pl.ds(c * GR, GR)],
                                    idx.at[pl.ds(0, GR)])
                    for j2 in range(GR):
                        pltpu.sync_copy(patb.at[pl.ds(0, 128)],
                                        acc.at[idx.at[j2]], add=True)

                return carry

            lax.fori_loop(0, NCH, body, 0)

            @pl.when(sid == 0)
            def _():
                pltpu.sync_copy(dst_hbm.at[pl.ds(FC * 8, 2)],
                                idx.at[pl.ds(0, 2)])
                for j2 in range(2):
                    pltpu.sync_copy(patb.at[pl.ds(0, 128)],
                                    acc.at[idx.at[j2]], add=True)

        @pl.when(cid == 1)
        def _():
            def nbody(nl_hbm, pofs):
                def body(j, carry):
                    c = j * NTILE + sid

                    @pl.when(c < NL2D)
                    def _():
                        _load_list_row(nl_hbm, lbuf, idx, 0, c, DUMMY)
                        pltpu.sync_copy(patb.at[pl.ds(pofs, 128)],
                                        acc.at[idx.at[0]], add=True)

                    return carry

                lax.fori_loop(0, NCHN, body, 0)

            nbody(inp_hbm, 128)
            nbody(outl_hbm, 256)

        plsc.subcore_barrier()
        row0 = sid * ACC_RPT

        @pl.when(cid == 0)
        def _():
            pltpu.sync_copy(acc.at[pl.ds(row0, ACC_RPT)],
                            o0_hbm.at[pl.ds(row0, ACC_RPT)])

        @pl.when(cid == 1)
        def _():
            pltpu.sync_copy(acc.at[pl.ds(row0, ACC_RPT)],
                            o1_hbm.at[pl.ds(row0, ACC_RPT)])

    return k(dst2d, inp_l, out_l, pats, zrows8, dep)


def _node_body(nfi_g, q0, q1, q2, q3, nfo_g, f1sa, f1sb, f2m_g,
               cnt_b, cnt16_b, cnto_ob16, cnti_ib16, cnto_ib16, validm16,
               wi1a, wq0, wq1, wq2, wq3, bi1, wi2, bi2, wi3, bi3,
               wi4a, bi4a, wi4b, bi4b,
               wo1a, wo1ba, wo1bb, wo1c, bo1, wo2, bo2, wo3, bo3,
               wo4a, bo4a, wo4b, bo4b,
               eia_o, eib_o, eoa_o, eob_o):
    x = (nfi_g[...] @ wi1a[...] + q0[...] @ wq0[...] + q1[...] @ wq1[...]
         + q2[...] @ wq2[...] + q3[...] @ wq3[...] + bi1[...])
    x = _leaky(x)
    x = _leaky(x @ wi2[...] + bi2[...])
    x = _leaky(x @ wi3[...] + bi3[...])
    keep16 = jnp.where(cnto_ib16[...] > 0.0, 0.0, 1.0)
    fi16 = validm16[...] * keep16 / jnp.maximum(cnti_ib16[...], 1.0)
    eia_o[...] = (x @ wi4a[...] + bi4a[...]) * fi16
    eib_o[...] = (x @ wi4b[...] + bi4b[...]) * fi16

    cb = cnt_b[...]
    inv16 = 1.0 / jnp.maximum(cnt16_b[...], 1.0)
    nfo2 = jnp.where(cb > 0.0, f2m_g[...], 0.0)
    y = (nfo_g[...] @ wo1a[...]
         + (f1sa[...] * inv16) @ wo1ba[...]
         + (f1sb[...] * inv16) @ wo1bb[...]
         + nfo2 @ wo1c[...] + bo1[...])
    y = _leaky(y)
    y = _leaky(y @ wo2[...] + bo2[...])
    y = _leaky(y @ wo3[...] + bo3[...])
    fo16 = validm16[...] / jnp.maximum(cnto_ob16[...], 1.0)
    eoa_o[...] = (y @ wo4a[...] + bo4a[...]) * fo16
    eob_o[...] = (y @ wo4b[...] + bo4b[...]) * fo16


def _node_mlps(nf_i, nfi_q, nf_o, f1s_h, f2m_o,
               cnt_b, cnt_b16, cnto_ob16, cnti_ib16, cnto_ib16, validm16,
               pi, po):
    n4 = NPAD // PACK
    (wi1, bi1), (wi2, bi2), (wi3, bi3), (wi4, bi4) = pi
    (wo1, bo1), (wo2, bo2), (wo3, bo3), (wo4, bo4) = po
    args = (nf_i.reshape(n4, 128),
            nfi_q[0].reshape(n4, 64), nfi_q[1].reshape(n4, 64),
            nfi_q[2].reshape(n4, 64), nfi_q[3].reshape(n4, 64),
            nf_o.reshape(n4, 128),
            f1s_h[0].reshape(n4, 64), f1s_h[1].reshape(n4, 64),
            f2m_o.reshape(n4, 128),
            cnt_b.reshape(n4, 128), cnt_b16.reshape(n4, 64),
            cnto_ob16.reshape(n4, 64), cnti_ib16.reshape(n4, 64),
            cnto_ib16.reshape(n4, 64), validm16.reshape(n4, 64),
            _bd(wi1[:32]),
            _bd(wi1[32:48]), _bd(wi1[48:64]), _bd(wi1[64:80]),
            _bd(wi1[80:96]), _bt(bi1),
            _bd(wi2), _bt(bi2), _bd(wi3), _bt(bi3),
            _bd(wi4[:, :16]), _bt(bi4[:16]), _bd(wi4[:, 16:]), _bt(bi4[16:]),
            _bd(wo1[:32]), _bd(wo1[32:48]), _bd(wo1[48:64]),
            _bd(wo1[64:96]), _bt(bo1),
            _bd(wo2), _bt(bo2), _bd(wo3), _bt(bo3),
            _bd(wo4[:, :16]), _bt(bo4[:16]), _bd(wo4[:, 16:]), _bt(bo4[16:]))
    BR = n4 // 7
    in_specs = []
    for ai, a in enumerate(args):
        if ai < 15:
            in_specs.append(
                pl.BlockSpec((BR, a.shape[1]), lambda i: (i, 0)))
        else:
            in_specs.append(pl.BlockSpec(a.shape, lambda i: (0, 0)))
    outs = pl.pallas_call(
        _node_body,
        grid=(7,),
        in_specs=in_specs,
        out_specs=[pl.BlockSpec((BR, 64), lambda i: (i, 0))] * 4,
        out_shape=[jax.ShapeDtypeStruct((n4, 64), jnp.float32)] * 4,
    )(*args)
    return [o.reshape(NPAD, 16) for o in outs]


def kernel(nf, ef_out, ef_in, params, edge_index_out, edge_index_in,
           input_nodes, output_nodes):
    n = nf.shape[0]
    src_o, dst_o = edge_index_out[0], edge_index_out[1]
    src_i, dst_i = edge_index_in[0], edge_index_in[1]

    gso, gdo = nf[src_o], nf[dst_o]
    gsi, gdi = nf[src_i], nf[dst_i]
    efi_q = _edge_mlp_out(gso, gdo, ef_out, params['msg_o2i'])
    f1a, f1b, f2 = _edge_mlp_in(gsi, gdi, ef_in, params['msg_i2o'])

    nfi_q = [jax.ops.segment_sum(q, dst_o, num_segments=n) for q in efi_q]
    f1s_h = [jax.ops.segment_sum(h, dst_i, num_segments=n)
             for h in (f1a, f1b)]
    ones = jnp.ones((E_EDGES,), dtype=jnp.float32)
    cnt = jax.ops.segment_sum(ones, dst_i, num_segments=n)
    f2max = jax.ops.segment_max(f2, dst_i, num_segments=n)

    pad0 = NPAD - input_nodes.shape[0]
    inp_g = jnp.pad(input_nodes, (0, pad0))
    out_g = jnp.pad(output_nodes, (0, pad0))
    o32 = jnp.ones((NPAD, 32), jnp.float32)
    o16 = jnp.ones((NPAD, 16), jnp.float32)
    eia, eib, eoa, eob = _node_mlps(
        nf[inp_g], [q[inp_g] for q in nfi_q],
        nf[out_g], [h[out_g] for h in f1s_h], f2max[out_g],
        jnp.broadcast_to(cnt[out_g][:, None], (NPAD, 32)),
        jnp.broadcast_to(cnt[out_g][:, None], (NPAD, 16)),
        o16, o16, 0.0 * o16, o16,
        params['reduce_i'], params['reduce_o'])

    ei = jnp.concatenate([eia, eib], axis=1)[:input_nodes.shape[0]]
    eo = jnp.concatenate([eoa, eob], axis=1)[:output_nodes.shape[0]]
    new_nf = jnp.zeros((n, 32), jnp.float32)
    new_nf = new_nf.at[input_nodes].set(ei)
    new_nf = new_nf.at[output_nodes].set(eo)
    return new_nf
